# baseline (speedup 1.0000x reference)
import functools

import jax
import jax.numpy as jnp
from jax import lax
from jax.experimental import pallas as pl
from jax.experimental.pallas import tpu as pltpu
from jax.experimental.pallas import tpu_sc as plsc

F = 26
D = 16
B = 16384
IN_DIM = F * D
P_DIM = 512
FP = 32
R = B * F
R2 = B * FP

NC = 2
NS = 16
NW = NC * NS
GSZ = 128
KG = 8
STEP = KG * GSZ
E_PER_W = R2 // NW
E_STEPS = E_PER_W // STEP
B_PER_W = R // NW
B_STEPS = B_PER_W // STEP


def _sc_gather(permidx2, feat2, emb_table, bias_flat):
    mesh = plsc.VectorSubcoreMesh(core_axis_name="c", subcore_axis_name="s")

    @functools.partial(
        pl.kernel,
        mesh=mesh,
        compiler_params=pltpu.CompilerParams(use_tc_tiling_on_sc=False),
        out_type=[
            jax.ShapeDtypeStruct((R2, D), jnp.float32),
            jax.ShapeDtypeStruct((R,), jnp.float32),
        ],
        scratch_types=[
            pltpu.VMEM((KG, GSZ), jnp.int32),
            pltpu.VMEM((STEP, D), jnp.float32),
            pltpu.VMEM((STEP,), jnp.float32),
            pltpu.SemaphoreType.DMA,
        ],
    )
    def k(perm_hbm, feat_hbm, emb_hbm, bias_hbm, rows_out, bias_out,
          idx_v, rows_v, brows_v, sem):
        wid = lax.axis_index("s") * NC + lax.axis_index("c")

        erow_base = wid * E_PER_W
        eblk_base = wid * (E_PER_W // GSZ)

        def ebody(g, carry):
            pltpu.sync_copy(perm_hbm.at[pl.ds(eblk_base + g * KG, KG)], idx_v)
            waits = []
            for j in range(KG):
                waits.append(pltpu.async_copy(
                    emb_hbm.at[idx_v.at[j]],
                    rows_v.at[pl.ds(j * GSZ, GSZ)], sem))
            for w in waits:
                w.wait()
            pltpu.sync_copy(rows_v, rows_out.at[pl.ds(erow_base + g * STEP, STEP)])
            return carry

        lax.fori_loop(0, E_STEPS, ebody, 0)

        brow_base = wid * B_PER_W
        bblk_base = wid * (B_PER_W // GSZ)

        def bbody(g, carry):
            pltpu.sync_copy(feat_hbm.at[pl.ds(bblk_base + g * KG, KG)], idx_v)
            waits = []
            for j in range(KG):
                waits.append(pltpu.async_copy(
                    bias_hbm.at[idx_v.at[j]],
                    brows_v.at[pl.ds(j * GSZ, GSZ)], sem))
            for w in waits:
                w.wait()
            pltpu.sync_copy(brows_v, bias_out.at[pl.ds(brow_base + g * STEP, STEP)])
            return carry

        lax.fori_loop(0, B_STEPS, bbody, 0)

    return k(permidx2, feat2, emb_table, bias_flat)


def _tc_body(raw0, raw1, raw2, raw3, fv_ref, bg_ref, e_ref, s_ref, w1_ref,
             b1_ref, w2_ref, b2_ref, wp2_ref, wp0_ref, wp1_ref, bp_ref, out_ref):
    fv = fv_ref[...]
    fve = jnp.dot(fv, e_ref[...], preferred_element_type=jnp.float32)
    scaled = jnp.concatenate(
        [raw0[...] * fve[:, 0:128],
         raw1[...] * fve[:, 128:256],
         raw2[...] * fve[:, 256:384],
         raw3[...] * fve[:, 384:512]], axis=1)
    s = jnp.dot(scaled, s_ref[...], preferred_element_type=jnp.float32)
    second = 0.5 * (jnp.sum(s * s, axis=1, keepdims=True)
                    - jnp.sum(scaled * scaled, axis=1, keepdims=True))
    first = jnp.sum(bg_ref[...] * fv, axis=1, keepdims=True)
    h = jnp.maximum(jnp.dot(scaled, w1_ref[...],
                            preferred_element_type=jnp.float32) + b1_ref[...], 0.0)
    y = jnp.maximum(jnp.dot(h, w2_ref[...],
                            preferred_element_type=jnp.float32) + b2_ref[...], 0.0)
    o = (jnp.dot(y, wp2_ref[...], preferred_element_type=jnp.float32)
         + first * wp0_ref[0, 0] + second * wp1_ref[0, 0] + bp_ref[0, 0])
    out_ref[...] = o


def _tc_compute(raw4, fv, biasg, E, S, W1P, b1P, W2P, b2P, Wp2P, wp0, wp1, bp):
    bm = 512
    nb = B // bm
    grid = (nb,)
    fixed = lambda i: (0, 0)

    def chunk_spec(c):
        return pl.BlockSpec((bm, 128), lambda i, c=c: (c * nb + i, 0))

    return pl.pallas_call(
        _tc_body,
        grid=grid,
        in_specs=[
            chunk_spec(0), chunk_spec(1), chunk_spec(2), chunk_spec(3),
            pl.BlockSpec((bm, F), lambda i: (i, 0)),
            pl.BlockSpec((bm, F), lambda i: (i, 0)),
            pl.BlockSpec((F, P_DIM), fixed),
            pl.BlockSpec((P_DIM, D), fixed),
            pl.BlockSpec((P_DIM, P_DIM), fixed),
            pl.BlockSpec((1, P_DIM), fixed),
            pl.BlockSpec((P_DIM, P_DIM), fixed),
            pl.BlockSpec((1, P_DIM), fixed),
            pl.BlockSpec((P_DIM, 1), fixed),
            pl.BlockSpec((1, 1), fixed),
            pl.BlockSpec((1, 1), fixed),
            pl.BlockSpec((1, 1), fixed),
        ],
        out_specs=pl.BlockSpec((bm, 1), lambda i: (i, 0)),
        out_shape=jax.ShapeDtypeStruct((B, 1), jnp.float32),
    )(raw4, raw4, raw4, raw4, fv, biasg, E, S, W1P, b1P, W2P,
      b2P, Wp2P, wp0, wp1, bp)


def kernel(features, feature_values, emb_table, bias_table, W1, b1, W2, b2, Wp, bp):
    featp = jnp.pad(features, ((0, 0), (0, FP - F)))
    perm = featp.reshape(B, 4, 8).transpose(1, 0, 2).reshape(-1)
    permidx2 = perm.reshape(R2 // GSZ, GSZ)
    feat2 = features.reshape(R // GSZ, GSZ)

    raw, biasg = _sc_gather(permidx2, feat2, emb_table, bias_table.reshape(-1))
    raw4 = raw.reshape(4 * B, FP // 4 * D)
    biasg = biasg.reshape(B, F)

    E = jnp.zeros((F, P_DIM), jnp.float32).at[:, :IN_DIM].set(
        jnp.kron(jnp.eye(F, dtype=jnp.float32), jnp.ones((1, D), jnp.float32)))
    S = jnp.zeros((P_DIM, D), jnp.float32).at[:IN_DIM].set(
        jnp.tile(jnp.eye(D, dtype=jnp.float32), (F, 1)))

    W1P = jnp.zeros((P_DIM, P_DIM), jnp.float32).at[:IN_DIM, :IN_DIM].set(W1)
    W2P = jnp.zeros((P_DIM, P_DIM), jnp.float32).at[:IN_DIM, :IN_DIM].set(W2)
    b1P = jnp.zeros((1, P_DIM), jnp.float32).at[:, :IN_DIM].set(b1[None, :])
    b2P = jnp.zeros((1, P_DIM), jnp.float32).at[:, :IN_DIM].set(b2[None, :])
    Wp2P = jnp.zeros((P_DIM, 1), jnp.float32).at[:IN_DIM].set(Wp[2:])

    out = _tc_compute(raw4, feature_values, biasg, E, S, W1P, b1P, W2P, b2P,
                      Wp2P, Wp[0:1], Wp[1:2], bp.reshape(1, 1))
    return out.reshape(-1)


# speedup vs baseline: 1.7210x; 1.7210x over previous
import functools

import jax
import jax.numpy as jnp
from jax import lax
from jax.experimental import pallas as pl
from jax.experimental.pallas import tpu as pltpu
from jax.experimental.pallas import tpu_sc as plsc

F = 26
D = 16
B = 16384
IN_DIM = F * D
P_DIM = 512
FP = 32
R = B * F
R2 = B * FP

NC = 2
NS = 16
NW = NC * NS
GSZ = 128
KG = 8
STEP = KG * GSZ
E_PER_W = R2 // NW
E_STEPS = E_PER_W // STEP
B_PER_W = R // NW
B_STEPS = B_PER_W // STEP


def _sc_gather(permidx2, feat2, emb_table, bias_flat):
    mesh = plsc.VectorSubcoreMesh(core_axis_name="c", subcore_axis_name="s")

    @functools.partial(
        pl.kernel,
        mesh=mesh,
        compiler_params=pltpu.CompilerParams(use_tc_tiling_on_sc=False),
        out_type=[
            jax.ShapeDtypeStruct((R2, D), jnp.float32),
            jax.ShapeDtypeStruct((R,), jnp.float32),
        ],
        scratch_types=[
            pltpu.VMEM((KG, GSZ), jnp.int32),
            pltpu.VMEM((STEP, D), jnp.float32),
            pltpu.VMEM((STEP,), jnp.float32),
            pltpu.SemaphoreType.DMA,
        ],
    )
    def k(perm_hbm, feat_hbm, emb_hbm, bias_hbm, rows_out, bias_out,
          idx_v, rows_v, brows_v, sem):
        wid = lax.axis_index("s") * NC + lax.axis_index("c")

        erow_base = wid * E_PER_W
        eblk_base = wid * (E_PER_W // GSZ)

        def ebody(g, carry):
            pltpu.sync_copy(perm_hbm.at[pl.ds(eblk_base + g * KG, KG)], idx_v)
            waits = []
            for j in range(KG):
                waits.append(pltpu.async_copy(
                    emb_hbm.at[idx_v.at[j]],
                    rows_v.at[pl.ds(j * GSZ, GSZ)], sem))
            for w in waits:
                w.wait()
            pltpu.sync_copy(rows_v, rows_out.at[pl.ds(erow_base + g * STEP, STEP)])
            return carry

        lax.fori_loop(0, E_STEPS, ebody, 0)

        brow_base = wid * B_PER_W
        bblk_base = wid * (B_PER_W // GSZ)

        def bbody(g, carry):
            pltpu.sync_copy(feat_hbm.at[pl.ds(bblk_base + g * KG, KG)], idx_v)
            waits = []
            for j in range(KG):
                waits.append(pltpu.async_copy(
                    bias_hbm.at[idx_v.at[j]],
                    brows_v.at[pl.ds(j * GSZ, GSZ)], sem))
            for w in waits:
                w.wait()
            pltpu.sync_copy(brows_v, bias_out.at[pl.ds(brow_base + g * STEP, STEP)])
            return carry

        lax.fori_loop(0, B_STEPS, bbody, 0)

    return k(permidx2, feat2, emb_table, bias_flat)


def _tc_body(raw0, raw1, raw2, raw3, fv_ref, bg_ref, e_ref, s_ref, w1_ref,
             b1_ref, w2_ref, b2_ref, wp2_ref, wp0_ref, wp1_ref, bp_ref, out_ref):
    fv = fv_ref[...]
    fve = jnp.dot(fv, e_ref[...], preferred_element_type=jnp.float32)
    scaled = jnp.concatenate(
        [raw0[...] * fve[:, 0:128],
         raw1[...] * fve[:, 128:256],
         raw2[...] * fve[:, 256:384],
         raw3[...] * fve[:, 384:512]], axis=1)
    s = jnp.dot(scaled, s_ref[...], preferred_element_type=jnp.float32)
    second = 0.5 * (jnp.sum(s * s, axis=1, keepdims=True)
                    - jnp.sum(scaled * scaled, axis=1, keepdims=True))
    first = jnp.sum(bg_ref[...] * fv, axis=1, keepdims=True)
    h = jnp.maximum(jnp.dot(scaled, w1_ref[...],
                            preferred_element_type=jnp.float32) + b1_ref[...], 0.0)
    y = jnp.maximum(jnp.dot(h, w2_ref[...],
                            preferred_element_type=jnp.float32) + b2_ref[...], 0.0)
    o = (jnp.dot(y, wp2_ref[...], preferred_element_type=jnp.float32)
         + first * wp0_ref[0, 0] + second * wp1_ref[0, 0] + bp_ref[0, 0])
    out_ref[...] = o


def _tc_compute(raw4, fv, biasg, E, S, W1P, b1P, W2P, b2P, Wp2P, wp0, wp1, bp):
    bm = 512
    nb = B // bm
    grid = (nb,)
    fixed = lambda i: (0, 0)

    def chunk_spec(c):
        return pl.BlockSpec((bm, 128), lambda i, c=c: (c * nb + i, 0))

    return pl.pallas_call(
        _tc_body,
        grid=grid,
        in_specs=[
            chunk_spec(0), chunk_spec(1), chunk_spec(2), chunk_spec(3),
            pl.BlockSpec((bm, F), lambda i: (i, 0)),
            pl.BlockSpec((bm, F), lambda i: (i, 0)),
            pl.BlockSpec((F, P_DIM), fixed),
            pl.BlockSpec((P_DIM, D), fixed),
            pl.BlockSpec((P_DIM, P_DIM), fixed),
            pl.BlockSpec((1, P_DIM), fixed),
            pl.BlockSpec((P_DIM, P_DIM), fixed),
            pl.BlockSpec((1, P_DIM), fixed),
            pl.BlockSpec((P_DIM, 1), fixed),
            pl.BlockSpec((1, 1), fixed),
            pl.BlockSpec((1, 1), fixed),
            pl.BlockSpec((1, 1), fixed),
        ],
        out_specs=pl.BlockSpec((bm, 1), lambda i: (i, 0)),
        out_shape=jax.ShapeDtypeStruct((B, 1), jnp.float32),
    )(raw4, raw4, raw4, raw4, fv, biasg, E, S, W1P, b1P, W2P,
      b2P, Wp2P, wp0, wp1, bp)


def kernel(features, feature_values, emb_table, bias_table, W1, b1, W2, b2, Wp, bp):
    featp = jnp.pad(features, ((0, 0), (0, FP - F)), mode="edge")
    perm = featp.reshape(B, 4, 8).transpose(1, 0, 2).reshape(-1)
    permidx2 = perm.reshape(R2 // GSZ, GSZ)
    feat2 = features.reshape(R // GSZ, GSZ)

    raw, biasg = _sc_gather(permidx2, feat2, emb_table, bias_table.reshape(-1))
    raw4 = raw.reshape(4 * B, FP // 4 * D)
    biasg = biasg.reshape(B, F)

    E = jnp.zeros((F, P_DIM), jnp.float32).at[:, :IN_DIM].set(
        jnp.kron(jnp.eye(F, dtype=jnp.float32), jnp.ones((1, D), jnp.float32)))
    S = jnp.zeros((P_DIM, D), jnp.float32).at[:IN_DIM].set(
        jnp.tile(jnp.eye(D, dtype=jnp.float32), (F, 1)))

    W1P = jnp.zeros((P_DIM, P_DIM), jnp.float32).at[:IN_DIM, :IN_DIM].set(W1)
    W2P = jnp.zeros((P_DIM, P_DIM), jnp.float32).at[:IN_DIM, :IN_DIM].set(W2)
    b1P = jnp.zeros((1, P_DIM), jnp.float32).at[:, :IN_DIM].set(b1[None, :])
    b2P = jnp.zeros((1, P_DIM), jnp.float32).at[:, :IN_DIM].set(b2[None, :])
    Wp2P = jnp.zeros((P_DIM, 1), jnp.float32).at[:IN_DIM].set(Wp[2:])

    out = _tc_compute(raw4, feature_values, biasg, E, S, W1P, b1P, W2P, b2P,
                      Wp2P, Wp[0:1], Wp[1:2], bp.reshape(1, 1))
    return out.reshape(-1)


# speedup vs baseline: 1.7253x; 1.0025x over previous
import functools

import jax
import jax.numpy as jnp
from jax import lax
from jax.experimental import pallas as pl
from jax.experimental.pallas import tpu as pltpu
from jax.experimental.pallas import tpu_sc as plsc

F = 26
D = 16
B = 16384
IN_DIM = F * D
P_DIM = 512
FP = 32
R = B * F
R2 = B * FP

NC = 2
NS = 16
NW = NC * NS
GSZ = 128
KG = 8
STEP = KG * GSZ
E_PER_W = R2 // NW
E_STEPS = E_PER_W // STEP
B_PER_W = R // NW
B_STEPS = B_PER_W // STEP


def _sc_gather(permidx2, feat2, emb_table, bias_flat):
    mesh = plsc.VectorSubcoreMesh(core_axis_name="c", subcore_axis_name="s")

    @functools.partial(
        pl.kernel,
        mesh=mesh,
        compiler_params=pltpu.CompilerParams(use_tc_tiling_on_sc=False),
        out_type=[
            jax.ShapeDtypeStruct((R2, D), jnp.float32),
            jax.ShapeDtypeStruct((R,), jnp.float32),
        ],
        scratch_types=[
            pltpu.VMEM((KG, GSZ), jnp.int32),
            pltpu.VMEM((STEP, D), jnp.float32),
            pltpu.VMEM((STEP,), jnp.float32),
            pltpu.SemaphoreType.DMA,
        ],
    )
    def k(perm_hbm, feat_hbm, emb_hbm, bias_hbm, rows_out, bias_out,
          idx_v, rows_v, brows_v, sem):
        wid = lax.axis_index("s") * NC + lax.axis_index("c")

        erow_base = wid * E_PER_W
        eblk_base = wid * (E_PER_W // GSZ)

        def ebody(g, carry):
            pltpu.sync_copy(perm_hbm.at[pl.ds(eblk_base + g * KG, KG)], idx_v)
            waits = []
            for j in range(KG):
                waits.append(pltpu.async_copy(
                    emb_hbm.at[idx_v.at[j]],
                    rows_v.at[pl.ds(j * GSZ, GSZ)], sem))
            for w in waits:
                w.wait()
            pltpu.sync_copy(rows_v, rows_out.at[pl.ds(erow_base + g * STEP, STEP)])
            return carry

        lax.fori_loop(0, E_STEPS, ebody, 0)

        brow_base = wid * B_PER_W
        bblk_base = wid * (B_PER_W // GSZ)

        def bbody(g, carry):
            pltpu.sync_copy(feat_hbm.at[pl.ds(bblk_base + g * KG, KG)], idx_v)
            waits = []
            for j in range(KG):
                waits.append(pltpu.async_copy(
                    bias_hbm.at[idx_v.at[j]],
                    brows_v.at[pl.ds(j * GSZ, GSZ)], sem))
            for w in waits:
                w.wait()
            pltpu.sync_copy(brows_v, bias_out.at[pl.ds(brow_base + g * STEP, STEP)])
            return carry

        lax.fori_loop(0, B_STEPS, bbody, 0)

    return k(permidx2, feat2, emb_table, bias_flat)


def _tc_body(raw0, raw1, raw2, raw3, fv_ref, bg_ref, e_ref, s_ref, w1_ref,
             b1_ref, w2_ref, b2_ref, wp2_ref, wp0_ref, wp1_ref, bp_ref, out_ref):
    fv = fv_ref[...]
    fve = jnp.dot(fv, e_ref[...], preferred_element_type=jnp.float32)
    scaled = jnp.concatenate(
        [raw0[...] * fve[:, 0:128],
         raw1[...] * fve[:, 128:256],
         raw2[...] * fve[:, 256:384],
         raw3[...] * fve[:, 384:512]], axis=1)
    s = jnp.dot(scaled, s_ref[...], preferred_element_type=jnp.float32)
    second = 0.5 * (jnp.sum(s * s, axis=1, keepdims=True)
                    - jnp.sum(scaled * scaled, axis=1, keepdims=True))
    first = jnp.sum(bg_ref[...] * fv, axis=1, keepdims=True)
    h = jnp.maximum(jnp.dot(scaled, w1_ref[...], precision=lax.Precision.DEFAULT,
                            preferred_element_type=jnp.float32) + b1_ref[...], 0.0)
    y = jnp.maximum(jnp.dot(h, w2_ref[...], precision=lax.Precision.DEFAULT,
                            preferred_element_type=jnp.float32) + b2_ref[...], 0.0)
    o = (jnp.dot(y, wp2_ref[...], preferred_element_type=jnp.float32)
         + first * wp0_ref[0, 0] + second * wp1_ref[0, 0] + bp_ref[0, 0])
    out_ref[...] = o


def _tc_compute(raw4, fv, biasg, E, S, W1P, b1P, W2P, b2P, Wp2P, wp0, wp1, bp):
    bm = 512
    nb = B // bm
    grid = (nb,)
    fixed = lambda i: (0, 0)

    def chunk_spec(c):
        return pl.BlockSpec((bm, 128), lambda i, c=c: (c * nb + i, 0))

    return pl.pallas_call(
        _tc_body,
        grid=grid,
        in_specs=[
            chunk_spec(0), chunk_spec(1), chunk_spec(2), chunk_spec(3),
            pl.BlockSpec((bm, F), lambda i: (i, 0)),
            pl.BlockSpec((bm, F), lambda i: (i, 0)),
            pl.BlockSpec((F, P_DIM), fixed),
            pl.BlockSpec((P_DIM, D), fixed),
            pl.BlockSpec((P_DIM, P_DIM), fixed),
            pl.BlockSpec((1, P_DIM), fixed),
            pl.BlockSpec((P_DIM, P_DIM), fixed),
            pl.BlockSpec((1, P_DIM), fixed),
            pl.BlockSpec((P_DIM, 1), fixed),
            pl.BlockSpec((1, 1), fixed),
            pl.BlockSpec((1, 1), fixed),
            pl.BlockSpec((1, 1), fixed),
        ],
        out_specs=pl.BlockSpec((bm, 1), lambda i: (i, 0)),
        out_shape=jax.ShapeDtypeStruct((B, 1), jnp.float32),
    )(raw4, raw4, raw4, raw4, fv, biasg, E, S, W1P, b1P, W2P,
      b2P, Wp2P, wp0, wp1, bp)


def kernel(features, feature_values, emb_table, bias_table, W1, b1, W2, b2, Wp, bp):
    featp = jnp.pad(features, ((0, 0), (0, FP - F)), mode="edge")
    perm = featp.reshape(B, 4, 8).transpose(1, 0, 2).reshape(-1)
    permidx2 = perm.reshape(R2 // GSZ, GSZ)
    feat2 = features.reshape(R // GSZ, GSZ)

    raw, biasg = _sc_gather(permidx2, feat2, emb_table, bias_table.reshape(-1))
    raw4 = raw.reshape(4 * B, FP // 4 * D)
    biasg = biasg.reshape(B, F)

    E = jnp.zeros((F, P_DIM), jnp.float32).at[:, :IN_DIM].set(
        jnp.kron(jnp.eye(F, dtype=jnp.float32), jnp.ones((1, D), jnp.float32)))
    S = jnp.zeros((P_DIM, D), jnp.float32).at[:IN_DIM].set(
        jnp.tile(jnp.eye(D, dtype=jnp.float32), (F, 1)))

    W1P = jnp.zeros((P_DIM, P_DIM), jnp.float32).at[:IN_DIM, :IN_DIM].set(W1)
    W2P = jnp.zeros((P_DIM, P_DIM), jnp.float32).at[:IN_DIM, :IN_DIM].set(W2)
    b1P = jnp.zeros((1, P_DIM), jnp.float32).at[:, :IN_DIM].set(b1[None, :])
    b2P = jnp.zeros((1, P_DIM), jnp.float32).at[:, :IN_DIM].set(b2[None, :])
    Wp2P = jnp.zeros((P_DIM, 1), jnp.float32).at[:IN_DIM].set(Wp[2:])

    out = _tc_compute(raw4, feature_values, biasg, E, S, W1P, b1P, W2P, b2P,
                      Wp2P, Wp[0:1], Wp[1:2], bp.reshape(1, 1))
    return out.reshape(-1)
